# baseline (device time: 33774 ns/iter reference)
import jax
import jax.numpy as jnp
from jax import lax
from jax.experimental import pallas as pl
from jax.experimental.pallas import tpu as pltpu

B, S, H_LOCAL, D = 4, 512, 8, 64
K = H_LOCAL * D
N = 1024
S_HALF = S // 2


def kernel(O, Wo):
    Or = O.reshape(B, S, K)

    def body(o_ref, w_ref, out_ref, send_buf, recv_buf, send_sem, recv_sem):
        my_x = lax.axis_index("x")
        my_y = lax.axis_index("y")
        peer = 1 - my_y

        barrier_sem = pltpu.get_barrier_semaphore()
        pl.semaphore_signal(
            barrier_sem, inc=1,
            device_id=(my_x, peer), device_id_type=pl.DeviceIdType.MESH,
        )
        pl.semaphore_wait(barrier_sem, 1)

        w = w_ref[:].astype(jnp.bfloat16)

        for b in range(B):
            ob = o_ref[b, pl.ds(peer * S_HALF, S_HALF), :].astype(jnp.bfloat16)
            send_buf[b] = jnp.dot(
                ob, w, preferred_element_type=jnp.float32
            ).astype(jnp.bfloat16)

        rdma = pltpu.make_async_remote_copy(
            src_ref=send_buf,
            dst_ref=recv_buf,
            send_sem=send_sem,
            recv_sem=recv_sem,
            device_id=(my_x, peer),
            device_id_type=pl.DeviceIdType.MESH,
        )
        rdma.start()

        for b in range(B):
            ob = o_ref[b, pl.ds(my_y * S_HALF, S_HALF), :].astype(jnp.bfloat16)
            out_ref[b] = jnp.dot(ob, w, preferred_element_type=jnp.float32)

        rdma.wait()

        for b in range(B):
            out_ref[b] = out_ref[b] + recv_buf[b].astype(jnp.float32)

    return pl.pallas_call(
        body,
        out_shape=jax.ShapeDtypeStruct((B, S_HALF, N), jnp.float32),
        in_specs=[
            pl.BlockSpec(memory_space=pltpu.VMEM),
            pl.BlockSpec(memory_space=pltpu.VMEM),
        ],
        out_specs=pl.BlockSpec(memory_space=pltpu.VMEM),
        scratch_shapes=[
            pltpu.VMEM((B, S_HALF, N), jnp.bfloat16),
            pltpu.VMEM((B, S_HALF, N), jnp.bfloat16),
            pltpu.SemaphoreType.DMA,
            pltpu.SemaphoreType.DMA,
        ],
        compiler_params=pltpu.CompilerParams(collective_id=0),
    )(Or, Wo)


# device time: 32532 ns/iter; 1.0382x vs baseline; 1.0382x over previous
import jax
import jax.numpy as jnp
from jax import lax
from jax.experimental import pallas as pl
from jax.experimental.pallas import tpu as pltpu

B, S, H_LOCAL, D = 4, 512, 8, 64
K = H_LOCAL * D
N = 1024
S_HALF = S // 2


def kernel(O, Wo):
    Or = O.reshape(B, S, K)

    def body(o_ref, w_ref, out_ref, send_buf, recv_buf, send_sems, recv_sems):
        my_x = lax.axis_index("x")
        my_y = lax.axis_index("y")
        peer = 1 - my_y

        barrier_sem = pltpu.get_barrier_semaphore()
        pl.semaphore_signal(
            barrier_sem, inc=1,
            device_id=(my_x, peer), device_id_type=pl.DeviceIdType.MESH,
        )
        pl.semaphore_wait(barrier_sem, 1)

        w = w_ref[:].astype(jnp.bfloat16)

        rdmas = []
        for b in range(B):
            ob = o_ref[b, pl.ds(peer * S_HALF, S_HALF), :].astype(jnp.bfloat16)
            send_buf[b] = jnp.dot(
                ob, w, preferred_element_type=jnp.float32
            ).astype(jnp.bfloat16)
            rdma = pltpu.make_async_remote_copy(
                src_ref=send_buf.at[b],
                dst_ref=recv_buf.at[b],
                send_sem=send_sems.at[b],
                recv_sem=recv_sems.at[b],
                device_id=(my_x, peer),
                device_id_type=pl.DeviceIdType.MESH,
            )
            rdma.start()
            rdmas.append(rdma)

        for b in range(B):
            ob = o_ref[b, pl.ds(my_y * S_HALF, S_HALF), :].astype(jnp.bfloat16)
            out_ref[b] = jnp.dot(ob, w, preferred_element_type=jnp.float32)

        for b in range(B):
            rdmas[b].wait()
            out_ref[b] = out_ref[b] + recv_buf[b].astype(jnp.float32)

    return pl.pallas_call(
        body,
        out_shape=jax.ShapeDtypeStruct((B, S_HALF, N), jnp.float32),
        in_specs=[
            pl.BlockSpec(memory_space=pltpu.VMEM),
            pl.BlockSpec(memory_space=pltpu.VMEM),
        ],
        out_specs=pl.BlockSpec(memory_space=pltpu.VMEM),
        scratch_shapes=[
            pltpu.VMEM((B, S_HALF, N), jnp.bfloat16),
            pltpu.VMEM((B, S_HALF, N), jnp.bfloat16),
            pltpu.SemaphoreType.DMA((B,)),
            pltpu.SemaphoreType.DMA((B,)),
        ],
        compiler_params=pltpu.CompilerParams(collective_id=0),
    )(Or, Wo)


# device time: 29421 ns/iter; 1.1480x vs baseline; 1.1057x over previous
import jax
import jax.numpy as jnp
from jax import lax
from jax.experimental import pallas as pl
from jax.experimental.pallas import tpu as pltpu

B, S, H_LOCAL, D = 4, 512, 8, 64
K = H_LOCAL * D
N = 1024
S_HALF = S // 2


def kernel(O, Wo):
    Or = O.reshape(B, S, K)

    def body(o_ref, w_ref, out_ref, send_buf, recv_buf,
             y_send_sems, y_recv_sems, x_send_sems, x_recv_sems):
        my_x = lax.axis_index("x")
        my_y = lax.axis_index("y")
        peer_y = 1 - my_y
        peer_x = 1 - my_x

        barrier_sem = pltpu.get_barrier_semaphore()
        pl.semaphore_signal(
            barrier_sem, inc=1,
            device_id=(my_x, peer_y), device_id_type=pl.DeviceIdType.MESH,
        )
        pl.semaphore_signal(
            barrier_sem, inc=1,
            device_id=(peer_x, my_y), device_id_type=pl.DeviceIdType.MESH,
        )
        pl.semaphore_wait(barrier_sem, 2)

        w = w_ref[:].astype(jnp.bfloat16)

        y_rdmas = []
        for i in range(2):
            b = 2 * my_x + i
            ob = o_ref[pl.ds(b, 1), pl.ds(peer_y * S_HALF, S_HALF), :].reshape(
                S_HALF, K
            ).astype(jnp.bfloat16)
            send_buf[i] = jnp.dot(
                ob, w, preferred_element_type=jnp.float32
            ).astype(jnp.bfloat16)
            rdma = pltpu.make_async_remote_copy(
                src_ref=send_buf.at[pl.ds(i, 1)],
                dst_ref=recv_buf.at[pl.ds(b, 1)],
                send_sem=y_send_sems.at[i],
                recv_sem=y_recv_sems.at[i],
                device_id=(my_x, peer_y),
                device_id_type=pl.DeviceIdType.MESH,
            )
            rdma.start()
            y_rdmas.append(rdma)

        for b in range(B):
            ob = o_ref[b, pl.ds(my_y * S_HALF, S_HALF), :].astype(jnp.bfloat16)
            out_ref[b] = jnp.dot(ob, w, preferred_element_type=jnp.float32)

        x_rdmas = []
        for i in range(2):
            b = 2 * my_x + i
            y_rdmas[i].wait_recv()
            fwd = pltpu.make_async_remote_copy(
                src_ref=recv_buf.at[pl.ds(b, 1)],
                dst_ref=recv_buf.at[pl.ds(b, 1)],
                send_sem=x_send_sems.at[i],
                recv_sem=x_recv_sems.at[i],
                device_id=(peer_x, my_y),
                device_id_type=pl.DeviceIdType.MESH,
            )
            fwd.start()
            x_rdmas.append(fwd)
            out_ref[pl.ds(b, 1), :, :] = (
                out_ref[pl.ds(b, 1), :, :]
                + recv_buf[pl.ds(b, 1), :, :].astype(jnp.float32)
            )

        for i in range(2):
            bx = 2 * peer_x + i
            x_rdmas[i].wait()
            out_ref[pl.ds(bx, 1), :, :] = (
                out_ref[pl.ds(bx, 1), :, :]
                + recv_buf[pl.ds(bx, 1), :, :].astype(jnp.float32)
            )

        for i in range(2):
            y_rdmas[i].wait_send()

    return pl.pallas_call(
        body,
        out_shape=jax.ShapeDtypeStruct((B, S_HALF, N), jnp.float32),
        in_specs=[
            pl.BlockSpec(memory_space=pltpu.VMEM),
            pl.BlockSpec(memory_space=pltpu.VMEM),
        ],
        out_specs=pl.BlockSpec(memory_space=pltpu.VMEM),
        scratch_shapes=[
            pltpu.VMEM((2, S_HALF, N), jnp.bfloat16),
            pltpu.VMEM((B, S_HALF, N), jnp.bfloat16),
            pltpu.SemaphoreType.DMA((2,)),
            pltpu.SemaphoreType.DMA((2,)),
            pltpu.SemaphoreType.DMA((2,)),
            pltpu.SemaphoreType.DMA((2,)),
        ],
        compiler_params=pltpu.CompilerParams(collective_id=0),
    )(Or, Wo)


# device time: 26473 ns/iter; 1.2758x vs baseline; 1.1114x over previous
import jax
import jax.numpy as jnp
from jax import lax
from jax.experimental import pallas as pl
from jax.experimental.pallas import tpu as pltpu

B, S, H_LOCAL, D = 4, 512, 8, 64
K = H_LOCAL * D
N = 1024
S_HALF = S // 2
R = 128
NSLOT = 2 * B
NC = 4


def kernel(O, Wo):
    Or = O.reshape(B, S, K)

    def body(o_ref, w_ref, out_ref, send_buf, recv_buf,
             y_send_sems, y_recv_sems, x_send_sems, x_recv_sems):
        my_x = lax.axis_index("x")
        my_y = lax.axis_index("y")
        peer_y = 1 - my_y
        peer_x = 1 - my_x

        barrier_sem = pltpu.get_barrier_semaphore()
        pl.semaphore_signal(
            barrier_sem, inc=1,
            device_id=(my_x, peer_y), device_id_type=pl.DeviceIdType.MESH,
        )
        pl.semaphore_signal(
            barrier_sem, inc=1,
            device_id=(peer_x, my_y), device_id_type=pl.DeviceIdType.MESH,
        )
        pl.semaphore_wait(barrier_sem, 2)

        w = w_ref[:].astype(jnp.bfloat16)

        y_rdmas = []
        for j in range(NC):
            b = j // 2
            row0 = peer_y * S_HALF + (j % 2) * R
            ob = o_ref[
                pl.ds(2 * my_x + b, 1), pl.ds(row0, R), :
            ].reshape(R, K).astype(jnp.bfloat16)
            send_buf[j] = jnp.dot(
                ob, w, preferred_element_type=jnp.float32
            ).astype(jnp.bfloat16)
            rdma = pltpu.make_async_remote_copy(
                src_ref=send_buf.at[pl.ds(j, 1)],
                dst_ref=recv_buf.at[pl.ds(4 * my_x + j, 1)],
                send_sem=y_send_sems.at[j],
                recv_sem=y_recv_sems.at[j],
                device_id=(my_x, peer_y),
                device_id_type=pl.DeviceIdType.MESH,
            )
            rdma.start()
            y_rdmas.append(rdma)

        for b in range(B):
            ob = o_ref[b, pl.ds(my_y * S_HALF, S_HALF), :].astype(jnp.bfloat16)
            acc = jnp.dot(ob, w, preferred_element_type=jnp.float32)
            out_ref[pl.ds(2 * b, 2)] = acc.reshape(2, R, N)

        x_rdmas = []
        for j in range(NC):
            s = 4 * my_x + j
            y_rdmas[j].wait_recv()
            fwd = pltpu.make_async_remote_copy(
                src_ref=recv_buf.at[pl.ds(s, 1)],
                dst_ref=recv_buf.at[pl.ds(s, 1)],
                send_sem=x_send_sems.at[j],
                recv_sem=x_recv_sems.at[j],
                device_id=(peer_x, my_y),
                device_id_type=pl.DeviceIdType.MESH,
            )
            fwd.start()
            x_rdmas.append(fwd)
            out_ref[pl.ds(s, 1)] = (
                out_ref[pl.ds(s, 1)]
                + recv_buf[pl.ds(s, 1)].astype(jnp.float32)
            )

        for j in range(NC):
            sx = 4 * peer_x + j
            x_rdmas[j].wait()
            out_ref[pl.ds(sx, 1)] = (
                out_ref[pl.ds(sx, 1)]
                + recv_buf[pl.ds(sx, 1)].astype(jnp.float32)
            )

        for j in range(NC):
            y_rdmas[j].wait_send()

    out = pl.pallas_call(
        body,
        out_shape=jax.ShapeDtypeStruct((NSLOT, R, N), jnp.float32),
        in_specs=[
            pl.BlockSpec(memory_space=pltpu.VMEM),
            pl.BlockSpec(memory_space=pltpu.VMEM),
        ],
        out_specs=pl.BlockSpec(memory_space=pltpu.VMEM),
        scratch_shapes=[
            pltpu.VMEM((NC, R, N), jnp.bfloat16),
            pltpu.VMEM((NSLOT, R, N), jnp.bfloat16),
            pltpu.SemaphoreType.DMA((NC,)),
            pltpu.SemaphoreType.DMA((NC,)),
            pltpu.SemaphoreType.DMA((NC,)),
            pltpu.SemaphoreType.DMA((NC,)),
        ],
        compiler_params=pltpu.CompilerParams(collective_id=0),
    )(Or, Wo)
    return out.reshape(B, S_HALF, N)
